# baseline (speedup 1.0000x reference)
import functools

import jax
import jax.numpy as jnp
from jax import lax
from jax.experimental import pallas as pl
from jax.experimental.pallas import tpu as pltpu
from jax.experimental.pallas import tpu_sc as plsc

_B = 16384
_G = 26
_D = 128
_BG = _B * _G
_NC = 2
_NS = 16
_NW = _NC * _NS
_CHUNK = 128
_NBANK = 4
_NSPLIT = 2
_BS = _B // _NSPLIT
_BGS = _BS * _G
_ROWS_W = _BGS // _NW
_NCHUNK = _ROWS_W // _CHUNK
_NITER = _NCHUNK // _NBANK
_MBLK = 1024


def _sc_gather(x2d, table):
    mesh = plsc.VectorSubcoreMesh(core_axis_name="c", subcore_axis_name="s")

    @functools.partial(
        pl.kernel,
        out_type=jax.ShapeDtypeStruct((_BGS, _D), jnp.float32),
        mesh=mesh,
        scratch_types=[
            pltpu.VMEM((_NCHUNK, _CHUNK), jnp.int32),
            pltpu.VMEM((_NBANK, _CHUNK, _D), jnp.float32),
            [pltpu.SemaphoreType.DMA] * _NBANK,
            [pltpu.SemaphoreType.DMA] * _NBANK,
        ],
    )
    def gather_kernel(x_hbm, tab_hbm, out_hbm, idx_v, rows_v, sem_g, sem_w):
        wid = lax.axis_index("s") * _NC + lax.axis_index("c")
        pltpu.sync_copy(x_hbm.at[wid], idx_v)
        row0 = wid * _ROWS_W

        def fire_gather(j, b):
            pltpu.async_copy(tab_hbm.at[idx_v.at[j]], rows_v.at[b], sem_g[b])

        def wait_gather(b):
            pltpu.make_async_copy(
                tab_hbm.at[pl.ds(0, _CHUNK)], rows_v.at[b], sem_g[b]).wait()

        def fire_write(j, b):
            pltpu.async_copy(
                rows_v.at[b],
                out_hbm.at[pl.ds(row0 + j * _CHUNK, _CHUNK)],
                sem_w[b])

        def wait_write(b):
            pltpu.make_async_copy(
                rows_v.at[b], out_hbm.at[pl.ds(0, _CHUNK)], sem_w[b]).wait()

        def turn(t, carry):
            for i in range(_NBANK):
                c = _NBANK * t + i

                @pl.when(t >= 1)
                def _():
                    wait_write(i)
                fire_gather(c, i)

                if i == 0:
                    @pl.when(t >= 1)
                    def _():
                        wait_gather(_NBANK - 1)
                        fire_write(_NBANK * t - 1, _NBANK - 1)
                else:
                    wait_gather(i - 1)
                    fire_write(c - 1, i - 1)
            return carry

        lax.fori_loop(0, _NITER, turn, 0)
        wait_gather(_NBANK - 1)
        fire_write(_NCHUNK - 1, _NBANK - 1)
        for i in range(_NBANK):
            wait_write(i)

    return gather_kernel(x2d, table)


def _mm_body(x_ref, w_ref, o_ref):
    acc = None
    for g in range(_G):
        xb = x_ref[g].astype(jnp.bfloat16)
        wb = w_ref[:, g * _D:(g + 1) * _D].astype(jnp.bfloat16)
        p = lax.dot_general(
            xb, wb, (((1,), (1,)), ((), ())),
            preferred_element_type=jnp.float32)
        acc = p if acc is None else acc + p
    o_ref[...] = acc


def _mm(emb_gm, w):
    emb3 = emb_gm.reshape(_G, _BS, _D)
    return pl.pallas_call(
        _mm_body,
        grid=(_BS // _MBLK,),
        in_specs=[
            pl.BlockSpec((_G, _MBLK, _D), lambda i: (0, i, 0)),
            pl.BlockSpec((_D, _G * _D), lambda i: (0, 0)),
        ],
        out_specs=pl.BlockSpec((_MBLK, _D), lambda i: (i, 0)),
        out_shape=jax.ShapeDtypeStruct((_BS, _D), jnp.float32),
    )(emb3, w)


def kernel(x, table, W):
    xt = x.T
    outs = []
    for c in range(_NSPLIT):
        x2d = xt[:, c * _BS:(c + 1) * _BS].reshape(_NW, _NCHUNK, _CHUNK)
        emb_gm = _sc_gather(x2d, table)
        outs.append(_mm(emb_gm, W))
    return jnp.concatenate(outs, axis=0)


# speedup vs baseline: 1.0533x; 1.0533x over previous
import functools

import jax
import jax.numpy as jnp
from jax import lax
from jax.experimental import pallas as pl
from jax.experimental.pallas import tpu as pltpu
from jax.experimental.pallas import tpu_sc as plsc

_B = 16384
_G = 26
_D = 128
_BG = _B * _G
_NC = 2
_NS = 16
_NW = _NC * _NS
_CHUNK = 128
_NBANK = 4
_ROWS_W = _BG // _NW
_NCHUNK = _ROWS_W // _CHUNK
_NITER = _NCHUNK // _NBANK
_MBLK = 1024

def _sc_gather(x3d, table):
    mesh = plsc.VectorSubcoreMesh(core_axis_name="c", subcore_axis_name="s")

    @functools.partial(
        pl.kernel,
        out_type=jax.ShapeDtypeStruct((_BG, _D), jnp.float32),
        mesh=mesh,
        scratch_types=[
            pltpu.VMEM((_NCHUNK, _CHUNK), jnp.int32),
            pltpu.VMEM((_NBANK, _CHUNK, _D), jnp.float32),
            [pltpu.SemaphoreType.DMA] * _NBANK,
            [pltpu.SemaphoreType.DMA] * _NBANK,
        ],
    )
    def gather_kernel(x_hbm, tab_hbm, out_hbm, idx_v, rows_v, sem_g, sem_w):
        wid = lax.axis_index("s") * _NC + lax.axis_index("c")
        pltpu.sync_copy(x_hbm.at[wid], idx_v)
        row0 = wid * _ROWS_W

        def fire_gather(j, b):
            pltpu.async_copy(tab_hbm.at[idx_v.at[j]], rows_v.at[b], sem_g[b])

        def wait_gather(b):
            pltpu.make_async_copy(
                tab_hbm.at[pl.ds(0, _CHUNK)], rows_v.at[b], sem_g[b]).wait()

        def fire_write(j, b):
            pltpu.async_copy(
                rows_v.at[b],
                out_hbm.at[pl.ds(row0 + j * _CHUNK, _CHUNK)],
                sem_w[b])

        def wait_write(b):
            pltpu.make_async_copy(
                rows_v.at[b], out_hbm.at[pl.ds(0, _CHUNK)], sem_w[b]).wait()

        def turn(t, carry):
            for i in range(_NBANK):
                c = _NBANK * t + i

                @pl.when(t >= 1)
                def _():
                    wait_write(i)
                fire_gather(c, i)

                if i == 0:
                    @pl.when(t >= 1)
                    def _():
                        wait_gather(_NBANK - 1)
                        fire_write(_NBANK * t - 1, _NBANK - 1)
                else:
                    wait_gather(i - 1)
                    fire_write(c - 1, i - 1)
            return carry

        lax.fori_loop(0, _NITER, turn, 0)
        wait_gather(_NBANK - 1)
        fire_write(_NCHUNK - 1, _NBANK - 1)
        for i in range(_NBANK):
            wait_write(i)

    return gather_kernel(x3d, table)


def _mm_body(x_ref, w_ref, o_ref):
    acc = None
    for g in range(_G):
        xb = x_ref[g].astype(jnp.bfloat16)
        wb = w_ref[:, g * _D:(g + 1) * _D].astype(jnp.bfloat16)
        p = lax.dot_general(
            xb, wb, (((1,), (1,)), ((), ())),
            preferred_element_type=jnp.float32)
        acc = p if acc is None else acc + p
    o_ref[...] = acc


def _mm(emb_gm, w):
    emb3 = emb_gm.reshape(_G, _B, _D)
    return pl.pallas_call(
        _mm_body,
        grid=(_B // _MBLK,),
        in_specs=[
            pl.BlockSpec((_G, _MBLK, _D), lambda i: (0, i, 0)),
            pl.BlockSpec((_D, _G * _D), lambda i: (0, 0)),
        ],
        out_specs=pl.BlockSpec((_MBLK, _D), lambda i: (i, 0)),
        out_shape=jax.ShapeDtypeStruct((_B, _D), jnp.float32),
    )(emb3, w)


def kernel(x, table, W):
    x3d = x.T.reshape(_NW, _NCHUNK, _CHUNK)
    emb_gm = _sc_gather(x3d, table)
    return _mm(emb_gm, W)
